# baseline (speedup 1.0000x reference)
import functools

import jax
import jax.numpy as jnp
from jax import lax
from jax.experimental import pallas as pl
from jax.experimental.pallas import tpu as pltpu
from jax.experimental.pallas import tpu_sc as plsc

SEQ = 512
BATCH = 512
DM = 512
NW = 32
BPW = BATCH // NW
LANES = 16


def _sc_body(x_hbm, emb_hbm, out_hbm, emb_v, buf_v, sem):
    c = lax.axis_index("c")
    s = lax.axis_index("s")
    wid = s * 2 + c
    b0 = wid * BPW
    pltpu.sync_copy(emb_hbm.at[pl.ds(b0, BPW), :], emb_v)

    def seq_step(si, carry):
        pltpu.async_copy(x_hbm.at[si, pl.ds(b0, BPW), :], buf_v, sem).wait()
        for i in range(BPW):
            for j in range(DM // LANES):
                sl = pl.ds(j * LANES, LANES)
                buf_v[i, sl] = buf_v[i, sl] + emb_v[i, sl]
        pltpu.sync_copy(buf_v, out_hbm.at[si, pl.ds(b0, BPW), :])
        return carry

    lax.fori_loop(0, SEQ, seq_step, 0)


def _sc_kernel(x, embedding):
    emb = embedding[:BATCH]
    mesh = plsc.VectorSubcoreMesh(core_axis_name="c", subcore_axis_name="s")
    k = functools.partial(
        pl.kernel,
        out_type=jax.ShapeDtypeStruct((SEQ, BATCH, DM), jnp.float32),
        mesh=mesh,
        scratch_types=[
            pltpu.VMEM((BPW, DM), jnp.float32),
            pltpu.VMEM((BPW, DM), jnp.float32),
            pltpu.SemaphoreType.DMA,
        ],
    )(_sc_body)
    return k(x, emb)


def kernel(x, embedding):
    return _sc_kernel(x, embedding)


# speedup vs baseline: 2.0279x; 2.0279x over previous
import functools

import jax
import jax.numpy as jnp
from jax import lax
from jax.experimental import pallas as pl
from jax.experimental.pallas import tpu as pltpu
from jax.experimental.pallas import tpu_sc as plsc

SEQ = 512
BATCH = 512
DM = 512
NW = 32
BPW = BATCH // NW
CSS = 2
NBUF = 4
T = SEQ // CSS
LANES = 16


def _sc_body(x_hbm, emb_hbm, out_hbm, emb_v, b0_v, b1_v, b2_v, b3_v,
             si0, si1, si2, si3, so0, so1, so2, so3):
    c = lax.axis_index("c")
    s = lax.axis_index("s")
    wid = s * 2 + c
    r0 = wid * BPW
    bufs = (b0_v, b1_v, b2_v, b3_v)
    sin = (si0, si1, si2, si3)
    sout = (so0, so1, so2, so3)

    pltpu.sync_copy(emb_hbm.at[pl.ds(r0, BPW), :], emb_v)

    def in_copy(t, b):
        return pltpu.make_async_copy(
            x_hbm.at[pl.ds(t * CSS, CSS), pl.ds(r0, BPW), :], bufs[b], sin[b])

    def out_copy(t, b):
        return pltpu.make_async_copy(
            bufs[b], out_hbm.at[pl.ds(t * CSS, CSS), pl.ds(r0, BPW), :],
            sout[b])

    def accumulate(b):
        buf = bufs[b]

        def row(r, carry):
            er = lax.rem(r, BPW)
            cs = lax.div(r, BPW)
            for j in range(DM // LANES):
                sl = pl.ds(j * LANES, LANES)
                plsc.addupdate(buf.at[cs, er, sl], emb_v[er, sl])
            return carry

        lax.fori_loop(0, CSS * BPW, row, 0)

    in_copy(0, 0).start()
    in_copy(1, 1).start()

    def outer(u, carry):
        for b in range(NBUF):
            t = u * NBUF + b
            nb = (b + 2) % NBUF

            @pl.when(t + 2 < T)
            def _prefetch():
                @pl.when(t >= 2)
                def _drain():
                    out_copy(t - 2, nb).wait()

                in_copy(t + 2, nb).start()

            in_copy(t, b).wait()
            accumulate(b)
            out_copy(t, b).start()
        return carry

    lax.fori_loop(0, T // NBUF, outer, 0)

    for t in range(T - 4, T):
        out_copy(t, t % NBUF).wait()


def _sc_kernel(x, embedding):
    mesh = plsc.VectorSubcoreMesh(core_axis_name="c", subcore_axis_name="s")
    buf_t = pltpu.VMEM((CSS, BPW, DM), jnp.float32)
    k = functools.partial(
        pl.kernel,
        out_type=jax.ShapeDtypeStruct((SEQ, BATCH, DM), jnp.float32),
        mesh=mesh,
        scratch_types=[
            pltpu.VMEM((BPW, DM), jnp.float32),
            buf_t, buf_t, buf_t, buf_t,
            pltpu.SemaphoreType.DMA, pltpu.SemaphoreType.DMA,
            pltpu.SemaphoreType.DMA, pltpu.SemaphoreType.DMA,
            pltpu.SemaphoreType.DMA, pltpu.SemaphoreType.DMA,
            pltpu.SemaphoreType.DMA, pltpu.SemaphoreType.DMA,
        ],
    )(_sc_body)
    return k(x, embedding)


def kernel(x, embedding):
    return _sc_kernel(x, embedding)


# speedup vs baseline: 4.4390x; 2.1889x over previous
import functools

import jax
import jax.numpy as jnp
from jax import lax
from jax.experimental import pallas as pl
from jax.experimental.pallas import tpu as pltpu
from jax.experimental.pallas import tpu_sc as plsc

SEQ = 512
BATCH = 512
DM = 512
NW = 32
BPW = BATCH // NW
CSS = 2
NBUF = 4
T = SEQ // CSS
LANES = 16


def _sc_body(x_hbm, emb_hbm, out_hbm, emb_v, b0_v, b1_v, b2_v, b3_v,
             si0, si1, si2, si3, so0, so1, so2, so3):
    c = lax.axis_index("c")
    s = lax.axis_index("s")
    wid = s * 2 + c
    r0 = wid * BPW
    bufs = (b0_v, b1_v, b2_v, b3_v)
    sin = (si0, si1, si2, si3)
    sout = (so0, so1, so2, so3)

    pltpu.sync_copy(emb_hbm.at[pl.ds(r0, BPW), :], emb_v)

    def in_copy(t, b):
        return pltpu.make_async_copy(
            x_hbm.at[pl.ds(t * CSS, CSS), pl.ds(r0, BPW), :], bufs[b], sin[b])

    def out_copy(t, b):
        return pltpu.make_async_copy(
            bufs[b], out_hbm.at[pl.ds(t * CSS, CSS), pl.ds(r0, BPW), :],
            sout[b])

    def accumulate(b):
        buf = bufs[b]

        @plsc.parallel_loop(0, CSS * BPW, unroll=4)
        def _row(r):
            er = lax.rem(r, BPW)
            cs = lax.div(r, BPW)
            for j in range(DM // LANES):
                sl = pl.ds(j * LANES, LANES)
                plsc.addupdate(buf.at[cs, er, sl], emb_v[er, sl])

    in_copy(0, 0).start()
    in_copy(1, 1).start()

    def outer(u, carry):
        for b in range(NBUF):
            t = u * NBUF + b
            nb = (b + 2) % NBUF

            @pl.when(t + 2 < T)
            def _prefetch():
                @pl.when(t >= 2)
                def _drain():
                    out_copy(t - 2, nb).wait()

                in_copy(t + 2, nb).start()

            in_copy(t, b).wait()
            accumulate(b)
            out_copy(t, b).start()
        return carry

    lax.fori_loop(0, T // NBUF, outer, 0)

    for t in range(T - 4, T):
        out_copy(t, t % NBUF).wait()


def _sc_kernel(x, embedding):
    mesh = plsc.VectorSubcoreMesh(core_axis_name="c", subcore_axis_name="s")
    buf_t = pltpu.VMEM((CSS, BPW, DM), jnp.float32)
    k = functools.partial(
        pl.kernel,
        out_type=jax.ShapeDtypeStruct((SEQ, BATCH, DM), jnp.float32),
        mesh=mesh,
        scratch_types=[
            pltpu.VMEM((BPW, DM), jnp.float32),
            buf_t, buf_t, buf_t, buf_t,
            pltpu.SemaphoreType.DMA, pltpu.SemaphoreType.DMA,
            pltpu.SemaphoreType.DMA, pltpu.SemaphoreType.DMA,
            pltpu.SemaphoreType.DMA, pltpu.SemaphoreType.DMA,
            pltpu.SemaphoreType.DMA, pltpu.SemaphoreType.DMA,
        ],
    )(_sc_body)
    return k(x, embedding)


def kernel(x, embedding):
    return _sc_kernel(x, embedding)
